# Initial kernel scaffold; baseline (speedup 1.0000x reference)
import functools

import jax
import jax.numpy as jnp
from jax import lax
from jax.experimental import pallas as pl
from jax.experimental.pallas import tpu as pltpu
from jax.experimental.pallas import tpu_sc as plsc

NUM_EMB = 1000000
D = 32

_info = plsc.get_sparse_core_info()
NC = _info.num_cores
NS = _info.num_subcores
NW = NC * NS

B = 16384 * 50
BPW = B // NW
CH = 128
NCH = BPW // CH


@functools.partial(
    pl.kernel,
    out_type=jax.ShapeDtypeStruct((B, D), jnp.float32),
    mesh=plsc.VectorSubcoreMesh(core_axis_name="c", subcore_axis_name="s"),
    scratch_types=[
        pltpu.VMEM((NCH, CH), jnp.int32),
        pltpu.VMEM((2, CH, D), jnp.float32),
        pltpu.SemaphoreType.DMA,
        pltpu.SemaphoreType.DMA,
    ],
)
def _emb_lookup(weight_hbm, idx_hbm, out_hbm, idx_v, rows_v, gsem, ssem):
    wid = lax.axis_index("s") * NC + lax.axis_index("c")
    base = wid * BPW

    pltpu.sync_copy(idx_hbm.at[wid], idx_v)

    def chunk(c, buf):
        pltpu.async_copy(weight_hbm.at[idx_v.at[c]], rows_v.at[buf], gsem).wait()
        out_rows = out_hbm.at[pl.ds(base + c * CH, CH)]
        return pltpu.async_copy(rows_v.at[buf], out_rows, ssem)

    def step(c, _):
        scat = chunk(c, lax.rem(c, 2))
        scat.wait()
        return 0

    lax.fori_loop(0, NCH, step, 0)


def kernel(x, weight):
    xb, xs = x.shape
    idx = x.reshape(NW, NCH, CH).astype(jnp.int32)
    out = _emb_lookup(weight, idx)
    return out.reshape(xb, xs, D)


# speedup vs baseline: 1.0241x; 1.0241x over previous
import functools

import jax
import jax.numpy as jnp
from jax import lax
from jax.experimental import pallas as pl
from jax.experimental.pallas import tpu as pltpu
from jax.experimental.pallas import tpu_sc as plsc

NUM_EMB = 1000000
D = 32

_info = plsc.get_sparse_core_info()
NC = _info.num_cores
NS = _info.num_subcores
NW = NC * NS

B = 16384 * 50
BPW = B // NW
CH = 128
NCH = BPW // CH


@functools.partial(
    pl.kernel,
    out_type=jax.ShapeDtypeStruct((B, D), jnp.float32),
    mesh=plsc.VectorSubcoreMesh(core_axis_name="c", subcore_axis_name="s"),
    scratch_types=[
        pltpu.VMEM((NCH, CH), jnp.int32),
        pltpu.VMEM((2, CH, D), jnp.float32),
        pltpu.SemaphoreType.DMA,
        pltpu.SemaphoreType.DMA,
    ],
    compiler_params=pltpu.CompilerParams(use_tc_tiling_on_sc=False),
)
def _emb_lookup(weight_hbm, idx_hbm, out_hbm, idx_v, rows_v, gsem, ssem):
    wid = lax.axis_index("s") * NC + lax.axis_index("c")
    base = wid * BPW

    pltpu.sync_copy(idx_hbm.at[wid], idx_v)

    def chunk(c, buf):
        pltpu.async_copy(weight_hbm.at[idx_v.at[c]], rows_v.at[buf], gsem).wait()
        out_rows = out_hbm.at[pl.ds(base + c * CH, CH)]
        return pltpu.async_copy(rows_v.at[buf], out_rows, ssem)

    def step(c, _):
        scat = chunk(c, lax.rem(c, 2))
        scat.wait()
        return 0

    lax.fori_loop(0, NCH, step, 0)


def kernel(x, weight):
    xb, xs = x.shape
    idx = x.reshape(NW, NCH, CH).astype(jnp.int32)
    out = _emb_lookup(weight, idx)
    return out.reshape(xb, xs, D)


# speedup vs baseline: 1.1138x; 1.0877x over previous
import functools

import jax
import jax.numpy as jnp
from jax import lax
from jax.experimental import pallas as pl
from jax.experimental.pallas import tpu as pltpu
from jax.experimental.pallas import tpu_sc as plsc

NUM_EMB = 1000000
D = 32

_info = plsc.get_sparse_core_info()
NC = _info.num_cores
NS = _info.num_subcores
NW = NC * NS

B = 16384 * 50
BPW = B // NW
CH = 128
NCH = BPW // CH
GPB = 4
SCH = GPB * CH
NBUF = 5
NSC = BPW // SCH


@functools.partial(
    pl.kernel,
    out_type=jax.ShapeDtypeStruct((B, D), jnp.float32),
    mesh=plsc.VectorSubcoreMesh(core_axis_name="c", subcore_axis_name="s"),
    scratch_types=[
        pltpu.VMEM((NCH, CH), jnp.int32),
        pltpu.VMEM((NBUF, SCH, D), jnp.float32),
        [pltpu.SemaphoreType.DMA] * NBUF,
        [pltpu.SemaphoreType.DMA] * NBUF,
    ],
    compiler_params=pltpu.CompilerParams(use_tc_tiling_on_sc=False),
)
def _emb_lookup(weight_hbm, idx_hbm, out_hbm, idx_v, rows_v, gsems, ssems):
    wid = lax.axis_index("s") * NC + lax.axis_index("c")
    base = wid * BPW

    pltpu.sync_copy(idx_hbm.at[wid], idx_v)

    def fire_gathers(c, b):
        for j in range(GPB):
            pltpu.async_copy(
                weight_hbm.at[idx_v.at[c * GPB + j]],
                rows_v.at[b, pl.ds(j * CH, CH)],
                gsems[b],
            )

    def wait_gathers(c, b):
        for j in range(GPB):
            pltpu.make_async_copy(
                weight_hbm.at[idx_v.at[c * GPB + j]],
                rows_v.at[b, pl.ds(j * CH, CH)],
                gsems[b],
            ).wait()

    def fire_scatter(c, b):
        return pltpu.async_copy(
            rows_v.at[b], out_hbm.at[pl.ds(base + c * SCH, SCH)], ssems[b]
        )

    def wait_scatter(c, b):
        pltpu.make_async_copy(
            rows_v.at[b], out_hbm.at[pl.ds(base + c * SCH, SCH)], ssems[b]
        ).wait()

    for c in range(NBUF - 1):
        fire_gathers(c, c)

    wait_gathers(0, 0)
    fire_scatter(0, 0)
    fire_gathers(NBUF - 1, NBUF - 1)

    def group(g, _):
        c0 = 1 + g * NBUF
        for u in range(NBUF):
            c = c0 + u
            b = (1 + u) % NBUF
            wait_gathers(c, b)
            fire_scatter(c, b)
            n = c + NBUF - 1
            bn = (b + NBUF - 1) % NBUF
            wait_scatter(c - 1, bn)
            fire_gathers(n, bn)
        return 0

    n_groups = (NSC - NBUF) // NBUF
    lax.fori_loop(0, n_groups, group, 0)

    for c in range(NSC - NBUF + 1, NSC):
        b = c % NBUF
        wait_gathers(c, b)
        fire_scatter(c, b)

    for c in range(NSC - NBUF, NSC):
        wait_scatter(c, c % NBUF)


def kernel(x, weight):
    xb, xs = x.shape
    idx = x.reshape(NW, NCH, CH).astype(jnp.int32)
    out = _emb_lookup(weight, idx)
    return out.reshape(xb, xs, D)


# speedup vs baseline: 1.5978x; 1.4345x over previous
import functools

import jax
import jax.numpy as jnp
from jax import lax
from jax.experimental import pallas as pl
from jax.experimental.pallas import tpu as pltpu
from jax.experimental.pallas import tpu_sc as plsc

NUM_EMB = 1000000
D = 32
S = 50
NB = 16384
LANE = 128
DT = D // 8

_info = plsc.get_sparse_core_info()
NC = _info.num_cores
NS = _info.num_subcores
NW = NC * NS

BLK = 512
NCHUNK = S * NB // BLK // NW
CPG = NB // LANE // 4


def _iota16():
    return lax.iota(jnp.int32, 16)


@functools.partial(
    pl.kernel,
    out_type=jax.ShapeDtypeStruct((S, DT, NB // LANE, 8, LANE), jnp.float32),
    mesh=plsc.VectorSubcoreMesh(core_axis_name="c", subcore_axis_name="s"),
    scratch_types=[
        pltpu.VMEM((2, 4, LANE), jnp.int32),
        pltpu.VMEM((2, BLK, D), jnp.float32),
        pltpu.VMEM((2, DT, 4, 8, LANE), jnp.float32),
        [pltpu.SemaphoreType.DMA] * 2,
        [pltpu.SemaphoreType.DMA] * 2,
        [pltpu.SemaphoreType.DMA] * 2,
    ],
    compiler_params=pltpu.CompilerParams(
        use_tc_tiling_on_sc=False, needs_layout_passes=False),
)
def _emb_lookup(weight_hbm, xt_hbm, out_hbm, idx_v, rows_v, tile_v,
                isems, gsems, ssems):
    wid = lax.axis_index("s") * NC + lax.axis_index("c")
    g0 = wid * NCHUNK

    def sblk(c):
        g = g0 + c
        return g // CPG, lax.rem(g, CPG)

    def fire_idx(c, p):
        s, blk = sblk(c)
        pltpu.async_copy(xt_hbm.at[s, pl.ds(blk * 4, 4)], idx_v.at[p], isems[p])

    def wait_idx(c, p):
        s, blk = sblk(c)
        pltpu.make_async_copy(
            xt_hbm.at[s, pl.ds(blk * 4, 4)], idx_v.at[p], isems[p]).wait()

    def fire_gathers(p):
        for j in range(4):
            pltpu.async_copy(
                weight_hbm.at[idx_v.at[p, j]],
                rows_v.at[p, pl.ds(j * LANE, LANE)],
                gsems[p],
            )

    def wait_gathers(p):
        for j in range(4):
            pltpu.make_async_copy(
                weight_hbm.at[idx_v.at[p, j]],
                rows_v.at[p, pl.ds(j * LANE, LANE)],
                gsems[p],
            ).wait()

    def fire_scatters(c, p):
        s, blk = sblk(c)
        for dt in range(DT):
            pltpu.async_copy(
                tile_v.at[p, dt],
                out_hbm.at[s, dt, pl.ds(blk * 4, 4)],
                ssems[p],
            )

    def wait_scatters(c, p):
        s, blk = sblk(c)
        for dt in range(DT):
            pltpu.make_async_copy(
                tile_v.at[p, dt],
                out_hbm.at[s, dt, pl.ds(blk * 4, 4)],
                ssems[p],
            ).wait()

    def transpose_chunk(p):
        rows = rows_v.at[p]
        iota = _iota16()

        def tbody(i, _):
            for u in range(2):
                g = 2 * i + u
                bidx = g * 16 + iota
                btl = g // 8
                lo = lax.rem(g, 8) * 16
                for d in range(D):
                    v = plsc.load_gather(rows, [bidx, jnp.full((16,), d, jnp.int32)])
                    tile_v[p, d // 8, btl, d % 8, pl.ds(lo, 16)] = v
            return 0

        lax.fori_loop(0, BLK // 32, tbody, 0)

    def body(c, p, first, last):
        if not last:
            fire_idx(c + 1, 1 - p)
        wait_gathers(p)
        if not first:
            wait_scatters(c - 2, p)
        if not last:
            wait_idx(c + 1, 1 - p)
            fire_gathers(1 - p)
        transpose_chunk(p)
        fire_scatters(c, p)

    fire_idx(0, 0)
    wait_idx(0, 0)
    fire_gathers(0)

    body(0, 0, True, False)
    body(1, 1, True, False)

    def group(g, _):
        c = 2 + 2 * g
        body(c, 0, False, False)
        body(c + 1, 1, False, False)
        return 0

    lax.fori_loop(0, (NCHUNK - 4) // 2, group, 0)

    body(NCHUNK - 2, 0, False, False)
    body(NCHUNK - 1, 1, False, True)

    wait_scatters(NCHUNK - 2, 0)
    wait_scatters(NCHUNK - 1, 1)


def kernel(x, weight):
    xt = x.T.reshape(S, NB // LANE, LANE)
    phys = _emb_lookup(weight, xt)
    return phys.transpose(2, 4, 0, 1, 3).reshape(NB, S, D)


# speedup vs baseline: 2.0681x; 1.2943x over previous
import functools

import jax
import jax.numpy as jnp
from jax import lax
from jax.experimental import pallas as pl
from jax.experimental.pallas import tpu as pltpu
from jax.experimental.pallas import tpu_sc as plsc

NUM_EMB = 1000000
D = 32
S = 50
NB = 16384
LANE = 128
DT = D // 8

_info = plsc.get_sparse_core_info()
NC = _info.num_cores
NS = _info.num_subcores
NW = NC * NS

BLK = 512
NCHUNK = S * NB // BLK // NW
CPG = NB // LANE // 4


def _iota16():
    return lax.iota(jnp.int32, 16)


@functools.partial(
    pl.kernel,
    out_type=jax.ShapeDtypeStruct((S, DT, NB // LANE, 8, LANE), jnp.float32),
    mesh=plsc.VectorSubcoreMesh(core_axis_name="c", subcore_axis_name="s"),
    scratch_types=[
        pltpu.VMEM((2, 4, LANE), jnp.int32),
        pltpu.VMEM((2, BLK, D), jnp.float32),
        pltpu.VMEM((2, DT, 4, 8, LANE), jnp.float32),
        [pltpu.SemaphoreType.DMA] * 2,
        [pltpu.SemaphoreType.DMA] * 2,
        [pltpu.SemaphoreType.DMA] * 2,
    ],
    compiler_params=pltpu.CompilerParams(
        use_tc_tiling_on_sc=False, needs_layout_passes=False),
)
def _emb_lookup(weight_hbm, xt_hbm, out_hbm, idx_v, rows_v, tile_v,
                isems, gsems, ssems):
    wid = lax.axis_index("s") * NC + lax.axis_index("c")
    g0 = wid * NCHUNK

    def sblk(c):
        g = g0 + c
        return g // CPG, lax.rem(g, CPG)

    def fire_idx(c, p):
        s, blk = sblk(c)
        pltpu.async_copy(xt_hbm.at[s, pl.ds(blk * 4, 4)], idx_v.at[p], isems[p])

    def wait_idx(c, p):
        s, blk = sblk(c)
        pltpu.make_async_copy(
            xt_hbm.at[s, pl.ds(blk * 4, 4)], idx_v.at[p], isems[p]).wait()

    def fire_gathers(p):
        for j in range(4):
            pltpu.async_copy(
                weight_hbm.at[idx_v.at[p, j]],
                rows_v.at[p, pl.ds(j * LANE, LANE)],
                gsems[p],
            )

    def wait_gathers(p):
        for j in range(4):
            pltpu.make_async_copy(
                weight_hbm.at[idx_v.at[p, j]],
                rows_v.at[p, pl.ds(j * LANE, LANE)],
                gsems[p],
            ).wait()

    def fire_scatters(c, p):
        s, blk = sblk(c)
        for dt in range(DT):
            pltpu.async_copy(
                tile_v.at[p, dt],
                out_hbm.at[s, dt, pl.ds(blk * 4, 4)],
                ssems[p],
            )

    def wait_scatters(c, p):
        s, blk = sblk(c)
        for dt in range(DT):
            pltpu.make_async_copy(
                tile_v.at[p, dt],
                out_hbm.at[s, dt, pl.ds(blk * 4, 4)],
                ssems[p],
            ).wait()

    def transpose_chunk(p):
        rows = rows_v.at[p]
        iota = _iota16()
        dsplat = [jnp.full((16,), d, jnp.int32) for d in range(D)]

        def tbody(i, _):
            for u in range(2):
                g = 2 * i + u
                bidx = g * 16 + iota
                btl = g // 8
                lo = lax.rem(g, 8) * 16
                vs = [plsc.load_gather(rows, [bidx, dsplat[d]]) for d in range(D)]
                for d in range(D):
                    tile_v[p, d // 8, btl, d % 8, pl.ds(lo, 16)] = vs[d]
            return 0

        lax.fori_loop(0, BLK // 32, tbody, 0)

    def body(c, p, first, last):
        if not last:
            fire_idx(c + 1, 1 - p)
        wait_gathers(p)
        if not first:
            wait_scatters(c - 2, p)
        if not last:
            wait_idx(c + 1, 1 - p)
            fire_gathers(1 - p)
        transpose_chunk(p)
        fire_scatters(c, p)

    fire_idx(0, 0)
    wait_idx(0, 0)
    fire_gathers(0)

    body(0, 0, True, False)
    body(1, 1, True, False)

    def group(g, _):
        c = 2 + 2 * g
        body(c, 0, False, False)
        body(c + 1, 1, False, False)
        return 0

    lax.fori_loop(0, (NCHUNK - 4) // 2, group, 0)

    body(NCHUNK - 2, 0, False, False)
    body(NCHUNK - 1, 1, False, True)

    wait_scatters(NCHUNK - 2, 0)
    wait_scatters(NCHUNK - 1, 1)


def kernel(x, weight):
    xt = x.T.reshape(S, NB // LANE, LANE)
    phys = _emb_lookup(weight, xt)
    return phys.transpose(2, 4, 0, 1, 3).reshape(NB, S, D)
